# Initial kernel scaffold; baseline (speedup 1.0000x reference)
import jax
import jax.numpy as jnp
from jax.experimental import pallas as pl


def kernel(x, edge_index, W1, b1, W2, b2):
    raise NotImplementedError("write your pallas kernel here")


# speedup vs baseline: 36.2133x; 36.2133x over previous
import functools

import jax
import jax.numpy as jnp
from jax import lax
from jax.experimental import pallas as pl
from jax.experimental.pallas import tpu as pltpu
from jax.experimental.pallas import tpu_sc as plsc

N = 10000
E = 320000
D = 16

NC = 2
NS = 16
NW = NC * NS
CH = 128
NCH = -(-E // (NW * CH))
EPW = NCH * CH
E_PAD = NW * EPW
N_PAD = 10112
ZROWS = N_PAD // NS

_mesh = plsc.VectorSubcoreMesh(core_axis_name="c", subcore_axis_name="s")
_f32 = jnp.float32


def _wid():
    return lax.axis_index("s") * NC + lax.axis_index("c")


def _deg_body(colw, ones, zeros, out, colv, onesv, acc, sem):
    c = lax.axis_index("c")
    s = lax.axis_index("s")
    pltpu.sync_copy(zeros.at[pl.ds(s * ZROWS, ZROWS)], acc.at[pl.ds(s * ZROWS, ZROWS)])
    pltpu.sync_copy(colw.at[_wid()], colv)
    pltpu.sync_copy(ones, onesv)
    plsc.subcore_barrier()

    def chunk(j, carry):
        pltpu.sync_copy(onesv, acc.at[colv.at[j]], add=True)
        return carry

    lax.fori_loop(0, NCH, chunk, 0)
    plsc.subcore_barrier()
    pltpu.sync_copy(acc.at[pl.ds(s * ZROWS, ZROWS)],
                    out.at[pl.ds(c * N_PAD + s * ZROWS, ZROWS)])


def _layer_body(roww, colw, ytab, zeros, out, rowv, colv, buf, acc, sem):
    c = lax.axis_index("c")
    s = lax.axis_index("s")
    pltpu.sync_copy(zeros.at[pl.ds(s * ZROWS, ZROWS)], acc.at[pl.ds(s * ZROWS, ZROWS)])
    pltpu.sync_copy(roww.at[_wid()], rowv)
    pltpu.sync_copy(colw.at[_wid()], colv)
    plsc.subcore_barrier()

    def chunk(j, carry):
        pltpu.async_copy(ytab.at[rowv.at[j]], buf, sem).wait()
        pltpu.sync_copy(buf, acc.at[colv.at[j]], add=True)
        return carry

    lax.fori_loop(0, NCH, chunk, 0)
    plsc.subcore_barrier()
    pltpu.sync_copy(acc.at[pl.ds(s * ZROWS, ZROWS)],
                    out.at[pl.ds(c * N_PAD + s * ZROWS, ZROWS)])


_deg_call = pl.kernel(
    _deg_body,
    out_type=jax.ShapeDtypeStruct((NC * N_PAD, D), _f32),
    mesh=_mesh,
    compiler_params=pltpu.CompilerParams(use_tc_tiling_on_sc=False),
    scratch_types=[
        pltpu.VMEM((NCH, CH), jnp.int32),
        pltpu.VMEM((CH, D), _f32),
        pltpu.VMEM_SHARED((N_PAD, D), _f32),
        pltpu.SemaphoreType.DMA,
    ],
)

_layer_call = pl.kernel(
    _layer_body,
    out_type=jax.ShapeDtypeStruct((NC * N_PAD, D), _f32),
    mesh=_mesh,
    compiler_params=pltpu.CompilerParams(use_tc_tiling_on_sc=False),
    scratch_types=[
        pltpu.VMEM((NCH, CH), jnp.int32),
        pltpu.VMEM((NCH, CH), jnp.int32),
        pltpu.VMEM((CH, D), _f32),
        pltpu.VMEM_SHARED((N_PAD, D), _f32),
        pltpu.SemaphoreType.DMA,
    ],
)


def _tc1_body(x_ref, w1_ref, degs_ref, xw_ref, y_ref, dis_ref):
    xw = jnp.dot(x_ref[...], w1_ref[...], preferred_element_type=_f32)
    deg = degs_ref[0:N, :] + degs_ref[N_PAD:N_PAD + N, :] + 1.0
    dis = lax.rsqrt(deg)
    xw_ref[...] = xw
    y_ref[...] = dis * xw
    dis_ref[...] = dis


def _tc2_body(acc_ref, xw_ref, dis_ref, w2_ref, b1_ref, y2_ref, hw2_ref):
    a = acc_ref[0:N, :] + acc_ref[N_PAD:N_PAD + N, :]
    dis = dis_ref[...]
    h = dis * a + dis * dis * xw_ref[...] + b1_ref[...]
    h = jnp.maximum(h, 0.0)
    hw2 = jnp.dot(h, w2_ref[...], preferred_element_type=_f32)
    y2_ref[...] = dis * hw2
    hw2_ref[...] = jnp.broadcast_to(hw2, (N, D))


def _tc3_body(acc_ref, hw2_ref, dis_ref, b2_ref, out_ref):
    a = acc_ref[0:N, :] + acc_ref[N_PAD:N_PAD + N, :]
    dis = dis_ref[...]
    o = dis * a + dis * dis * hw2_ref[...] + b2_ref[...]
    out_ref[...] = o[:, 0:1]


_sds = jax.ShapeDtypeStruct

_tc1_call = pl.pallas_call(
    _tc1_body,
    out_shape=(_sds((N, D), _f32), _sds((N, D), _f32), _sds((N, D), _f32)),
)

_tc2_call = pl.pallas_call(
    _tc2_body,
    out_shape=(_sds((N, D), _f32), _sds((N, D), _f32)),
)

_tc3_call = pl.pallas_call(
    _tc3_body,
    out_shape=_sds((N, 1), _f32),
)


def kernel(x, edge_index, W1, b1, W2, b2):
    ei = edge_index.astype(jnp.int32)
    pad = jnp.full((2, E_PAD - E), N, dtype=jnp.int32)
    eip = jnp.concatenate([ei, pad], axis=1)
    roww = eip[0].reshape(NW, NCH, CH)
    colw = eip[1].reshape(NW, NCH, CH)

    ones = jnp.ones((CH, D), dtype=_f32)
    zeros = jnp.zeros((N_PAD, D), dtype=_f32)

    degs = _deg_call(colw, ones, zeros)
    xw1, y1, dis = _tc1_call(x, W1, degs)

    y1p = jnp.concatenate([y1, jnp.zeros((N_PAD - N, D), dtype=_f32)], axis=0)
    acc1 = _layer_call(roww, colw, y1p, zeros)

    y2, hw2 = _tc2_call(acc1, xw1, dis, W2, b1.reshape(1, D))
    y2p = jnp.concatenate([y2, jnp.zeros((N_PAD - N, D), dtype=_f32)], axis=0)
    acc2 = _layer_call(roww, colw, y2p, zeros)

    return _tc3_call(acc2, hw2, dis, b2.reshape(1, 1))


# speedup vs baseline: 44.7013x; 1.2344x over previous
import functools

import jax
import jax.numpy as jnp
from jax import lax
from jax.experimental import pallas as pl
from jax.experimental.pallas import tpu as pltpu
from jax.experimental.pallas import tpu_sc as plsc

N = 10000
E = 320000
D = 16

NC = 2
NS = 16
NW = NC * NS
CH = 128
K = 4
NCH = 80
EPW = NCH * CH
E_PAD = NW * EPW
N_PAD = 10112
ZROWS = N_PAD // NS

_mesh = plsc.VectorSubcoreMesh(core_axis_name="c", subcore_axis_name="s")
_f32 = jnp.float32


def _wid():
    return lax.axis_index("s") * NC + lax.axis_index("c")


def _deg_body(colw, ones, zeros, out, colv, onesv, acc, sem):
    c = lax.axis_index("c")
    s = lax.axis_index("s")
    pltpu.sync_copy(zeros.at[pl.ds(s * ZROWS, ZROWS)], acc.at[pl.ds(s * ZROWS, ZROWS)])
    pltpu.sync_copy(colw.at[_wid()], colv)
    pltpu.sync_copy(ones, onesv)
    plsc.subcore_barrier()

    def chunk(blk, carry):
        j0 = blk * 8
        descs = [pltpu.async_copy(onesv, acc.at[colv.at[j0 + b]], sem, add=True)
                 for b in range(8)]
        for d in descs:
            d.wait()
        return carry

    lax.fori_loop(0, NCH // 8, chunk, 0)
    plsc.subcore_barrier()
    pltpu.sync_copy(acc.at[pl.ds(s * ZROWS, ZROWS)],
                    out.at[pl.ds(c * N_PAD + s * ZROWS, ZROWS)])


def _layer_body(roww, colw, ytab, zeros, out, rowv, colv, buf,
                acc, g0, g1, g2, g3):
    c = lax.axis_index("c")
    s = lax.axis_index("s")
    gsem = (g0, g1, g2, g3)
    pltpu.sync_copy(zeros.at[pl.ds(s * ZROWS, ZROWS)], acc.at[pl.ds(s * ZROWS, ZROWS)])
    pltpu.sync_copy(roww.at[_wid()], rowv)
    pltpu.sync_copy(colw.at[_wid()], colv)
    plsc.subcore_barrier()

    for b in range(K):
        pltpu.async_copy(ytab.at[rowv.at[b]], buf.at[b], gsem[b])

    def outer(blk, carry):
        j0 = blk * K
        for b in range(K):
            j = j0 + b
            pltpu.make_async_copy(ytab.at[rowv.at[j]], buf.at[b], gsem[b]).wait()
            pltpu.sync_copy(buf.at[b], acc.at[colv.at[j]], add=True)
            pltpu.async_copy(ytab.at[rowv.at[j + K]], buf.at[b], gsem[b])
        return carry

    lax.fori_loop(0, NCH // K - 1, outer, 0)
    j0 = NCH - K
    for b in range(K):
        j = j0 + b
        pltpu.make_async_copy(ytab.at[rowv.at[j]], buf.at[b], gsem[b]).wait()
        pltpu.sync_copy(buf.at[b], acc.at[colv.at[j]], add=True)
    plsc.subcore_barrier()
    pltpu.sync_copy(acc.at[pl.ds(s * ZROWS, ZROWS)],
                    out.at[pl.ds(c * N_PAD + s * ZROWS, ZROWS)])


_deg_call = pl.kernel(
    _deg_body,
    out_type=jax.ShapeDtypeStruct((NC * N_PAD, D), _f32),
    mesh=_mesh,
    compiler_params=pltpu.CompilerParams(use_tc_tiling_on_sc=False),
    scratch_types=[
        pltpu.VMEM((NCH, CH), jnp.int32),
        pltpu.VMEM((CH, D), _f32),
        pltpu.VMEM_SHARED((N_PAD, D), _f32),
        pltpu.SemaphoreType.DMA,
    ],
)

_layer_call = pl.kernel(
    _layer_body,
    out_type=jax.ShapeDtypeStruct((NC * N_PAD, D), _f32),
    mesh=_mesh,
    compiler_params=pltpu.CompilerParams(use_tc_tiling_on_sc=False),
    scratch_types=[
        pltpu.VMEM((NCH, CH), jnp.int32),
        pltpu.VMEM((NCH, CH), jnp.int32),
        pltpu.VMEM((K, CH, D), _f32),
        pltpu.VMEM_SHARED((N_PAD, D), _f32),
        pltpu.SemaphoreType.DMA,
        pltpu.SemaphoreType.DMA,
        pltpu.SemaphoreType.DMA,
        pltpu.SemaphoreType.DMA,
    ],
)


def _tc1_body(x_ref, w1_ref, degs_ref, xw_ref, y_ref, dis_ref):
    xw = jnp.dot(x_ref[...], w1_ref[...], preferred_element_type=_f32)
    deg = degs_ref[0:N, :] + degs_ref[N_PAD:N_PAD + N, :] + 1.0
    dis = lax.rsqrt(deg)
    xw_ref[...] = xw
    y_ref[...] = dis * xw
    dis_ref[...] = dis


def _tc2_body(acc_ref, xw_ref, dis_ref, w2_ref, b1_ref, y2_ref, hw2_ref):
    a = acc_ref[0:N, :] + acc_ref[N_PAD:N_PAD + N, :]
    dis = dis_ref[...]
    h = dis * a + dis * dis * xw_ref[...] + b1_ref[...]
    h = jnp.maximum(h, 0.0)
    hw2 = jnp.dot(h, w2_ref[...], preferred_element_type=_f32)
    y2_ref[...] = dis * hw2
    hw2_ref[...] = jnp.broadcast_to(hw2, (N, D))


def _tc3_body(acc_ref, hw2_ref, dis_ref, b2_ref, out_ref):
    a = acc_ref[0:N, :] + acc_ref[N_PAD:N_PAD + N, :]
    dis = dis_ref[...]
    o = dis * a + dis * dis * hw2_ref[...] + b2_ref[...]
    out_ref[...] = o[:, 0:1]


_sds = jax.ShapeDtypeStruct

_tc1_call = pl.pallas_call(
    _tc1_body,
    out_shape=(_sds((N, D), _f32), _sds((N, D), _f32), _sds((N, D), _f32)),
)

_tc2_call = pl.pallas_call(
    _tc2_body,
    out_shape=(_sds((N, D), _f32), _sds((N, D), _f32)),
)

_tc3_call = pl.pallas_call(
    _tc3_body,
    out_shape=_sds((N, 1), _f32),
)


def kernel(x, edge_index, W1, b1, W2, b2):
    ei = edge_index.astype(jnp.int32)
    pad = jnp.full((2, E_PAD - E), N, dtype=jnp.int32)
    eip = jnp.concatenate([ei, pad], axis=1)
    roww = eip[0].reshape(NW, NCH, CH)
    colw = eip[1].reshape(NW, NCH, CH)

    ones = jnp.ones((CH, D), dtype=_f32)
    zeros = jnp.zeros((N_PAD, D), dtype=_f32)

    degs = _deg_call(colw, ones, zeros)
    xw1, y1, dis = _tc1_call(x, W1, degs)

    y1p = jnp.concatenate([y1, jnp.zeros((N_PAD - N, D), dtype=_f32)], axis=0)
    acc1 = _layer_call(roww, colw, y1p, zeros)

    y2, hw2 = _tc2_call(acc1, xw1, dis, W2, b1.reshape(1, D))
    y2p = jnp.concatenate([y2, jnp.zeros((N_PAD - N, D), dtype=_f32)], axis=0)
    acc2 = _layer_call(roww, colw, y2p, zeros)

    return _tc3_call(acc2, hw2, dis, b2.reshape(1, 1))


# speedup vs baseline: 44.8367x; 1.0030x over previous
import functools

import jax
import jax.numpy as jnp
from jax import lax
from jax.experimental import pallas as pl
from jax.experimental.pallas import tpu as pltpu
from jax.experimental.pallas import tpu_sc as plsc

N = 10000
E = 320000
D = 16

NC = 2
NS = 16
NW = NC * NS
CH = 128
K = 8
L = 4
NCH = 80
EPW = NCH * CH
E_PAD = NW * EPW
N_PAD = 10112
ZROWS = N_PAD // NS

_mesh = plsc.VectorSubcoreMesh(core_axis_name="c", subcore_axis_name="s")
_f32 = jnp.float32


def _wid():
    return lax.axis_index("s") * NC + lax.axis_index("c")


def _deg_body(colw, ones, zeros, out, colv, onesv, acc, sem):
    c = lax.axis_index("c")
    s = lax.axis_index("s")
    pltpu.sync_copy(zeros.at[pl.ds(s * ZROWS, ZROWS)], acc.at[pl.ds(s * ZROWS, ZROWS)])
    pltpu.sync_copy(colw.at[_wid()], colv)
    pltpu.sync_copy(ones, onesv)
    plsc.subcore_barrier()

    def chunk(blk, carry):
        j0 = blk * 8
        descs = [pltpu.async_copy(onesv, acc.at[colv.at[j0 + b]], sem, add=True)
                 for b in range(8)]
        for d in descs:
            d.wait()
        return carry

    lax.fori_loop(0, NCH // 8, chunk, 0)
    plsc.subcore_barrier()
    pltpu.sync_copy(acc.at[pl.ds(s * ZROWS, ZROWS)],
                    out.at[pl.ds(c * N_PAD + s * ZROWS, ZROWS)])


def _layer_body(roww, colw, ytab, zeros, out, rowv, colv, buf,
                acc, gsem, ssem):
    c = lax.axis_index("c")
    s = lax.axis_index("s")
    pltpu.sync_copy(zeros.at[pl.ds(s * ZROWS, ZROWS)], acc.at[pl.ds(s * ZROWS, ZROWS)])
    pltpu.sync_copy(roww.at[_wid()], rowv)
    pltpu.sync_copy(colw.at[_wid()], colv)
    plsc.subcore_barrier()

    def fire_gather(j, b):
        pltpu.async_copy(ytab.at[rowv.at[j]], buf.at[b], gsem.at[b])

    def wait_gather(j, b):
        pltpu.make_async_copy(ytab.at[rowv.at[j]], buf.at[b], gsem.at[b]).wait()

    def fire_scatter(j, b):
        pltpu.async_copy(buf.at[b], acc.at[colv.at[j]], ssem.at[b], add=True)

    def wait_scatter(j, b):
        pltpu.make_async_copy(buf.at[b], acc.at[colv.at[j]], ssem.at[b]).wait()

    def step(j, jmod):
        b = jmod % K
        wait_gather(j, b)
        fire_scatter(j, b)
        jn, bn = j + L, (jmod + L) % K
        if not isinstance(jn, int) or jn < NCH:
            if not isinstance(jn, int) or jn >= K:
                wait_scatter(jn - K, bn)
            fire_gather(jn, bn)

    for j in range(L):
        fire_gather(j, j % K)

    def outer(blk, carry):
        j0 = blk * K
        for b in range(K):
            step(j0 + b, b)
        return carry

    for b in range(K):
        step(b, b)
    lax.fori_loop(1, NCH // K - 1, outer, 0)
    for b in range(K):
        step(NCH - K + b, b)
    for b in range(K):
        wait_scatter(NCH - K + b, b)
    plsc.subcore_barrier()
    pltpu.sync_copy(acc.at[pl.ds(s * ZROWS, ZROWS)],
                    out.at[pl.ds(c * N_PAD + s * ZROWS, ZROWS)])


_deg_call = pl.kernel(
    _deg_body,
    out_type=jax.ShapeDtypeStruct((NC * N_PAD, D), _f32),
    mesh=_mesh,
    compiler_params=pltpu.CompilerParams(use_tc_tiling_on_sc=False),
    scratch_types=[
        pltpu.VMEM((NCH, CH), jnp.int32),
        pltpu.VMEM((CH, D), _f32),
        pltpu.VMEM_SHARED((N_PAD, D), _f32),
        pltpu.SemaphoreType.DMA,
    ],
)

_layer_call = pl.kernel(
    _layer_body,
    out_type=jax.ShapeDtypeStruct((NC * N_PAD, D), _f32),
    mesh=_mesh,
    compiler_params=pltpu.CompilerParams(use_tc_tiling_on_sc=False),
    scratch_types=[
        pltpu.VMEM((NCH, CH), jnp.int32),
        pltpu.VMEM((NCH, CH), jnp.int32),
        pltpu.VMEM((K, CH, D), _f32),
        pltpu.VMEM_SHARED((N_PAD, D), _f32),
        pltpu.SemaphoreType.DMA((K,)),
        pltpu.SemaphoreType.DMA((K,)),
    ],
)


def _tc1_body(x_ref, w1_ref, degs_ref, xw_ref, y_ref, dis_ref):
    xw = jnp.dot(x_ref[...], w1_ref[...], preferred_element_type=_f32)
    deg = degs_ref[0:N, :] + degs_ref[N_PAD:N_PAD + N, :] + 1.0
    dis = lax.rsqrt(deg)
    xw_ref[...] = xw
    y_ref[...] = dis * xw
    dis_ref[...] = dis


def _tc2_body(acc_ref, xw_ref, dis_ref, w2_ref, b1_ref, y2_ref, hw2_ref):
    a = acc_ref[0:N, :] + acc_ref[N_PAD:N_PAD + N, :]
    dis = dis_ref[...]
    h = dis * a + dis * dis * xw_ref[...] + b1_ref[...]
    h = jnp.maximum(h, 0.0)
    hw2 = jnp.dot(h, w2_ref[...], preferred_element_type=_f32)
    y2_ref[...] = dis * hw2
    hw2_ref[...] = jnp.broadcast_to(hw2, (N, D))


def _tc3_body(acc_ref, hw2_ref, dis_ref, b2_ref, out_ref):
    a = acc_ref[0:N, :] + acc_ref[N_PAD:N_PAD + N, :]
    dis = dis_ref[...]
    o = dis * a + dis * dis * hw2_ref[...] + b2_ref[...]
    out_ref[...] = o[:, 0:1]


_sds = jax.ShapeDtypeStruct

_tc1_call = pl.pallas_call(
    _tc1_body,
    out_shape=(_sds((N, D), _f32), _sds((N, D), _f32), _sds((N, D), _f32)),
)

_tc2_call = pl.pallas_call(
    _tc2_body,
    out_shape=(_sds((N, D), _f32), _sds((N, D), _f32)),
)

_tc3_call = pl.pallas_call(
    _tc3_body,
    out_shape=_sds((N, 1), _f32),
)


def kernel(x, edge_index, W1, b1, W2, b2):
    ei = edge_index.astype(jnp.int32)
    pad = jnp.full((2, E_PAD - E), N, dtype=jnp.int32)
    eip = jnp.concatenate([ei, pad], axis=1)
    roww = eip[0].reshape(NW, NCH, CH)
    colw = eip[1].reshape(NW, NCH, CH)

    ones = jnp.ones((CH, D), dtype=_f32)
    zeros = jnp.zeros((N_PAD, D), dtype=_f32)

    degs = _deg_call(colw, ones, zeros)
    xw1, y1, dis = _tc1_call(x, W1, degs)

    y1p = jnp.concatenate([y1, jnp.zeros((N_PAD - N, D), dtype=_f32)], axis=0)
    acc1 = _layer_call(roww, colw, y1p, zeros)

    y2, hw2 = _tc2_call(acc1, xw1, dis, W2, b1.reshape(1, D))
    y2p = jnp.concatenate([y2, jnp.zeros((N_PAD - N, D), dtype=_f32)], axis=0)
    acc2 = _layer_call(roww, colw, y2p, zeros)

    return _tc3_call(acc2, hw2, dis, b2.reshape(1, 1))


# speedup vs baseline: 48.5187x; 1.0821x over previous
import functools

import jax
import jax.numpy as jnp
from jax import lax
from jax.experimental import pallas as pl
from jax.experimental.pallas import tpu as pltpu
from jax.experimental.pallas import tpu_sc as plsc

N = 10000
E = 320000
D = 16

NC = 2
NS = 16
NW = NC * NS
CH = 128
K = 8
L = 4
NCH = 80
EPW = NCH * CH
E_PAD = NW * EPW
N_PAD = 10112
ZROWS = N_PAD // NS

_mesh = plsc.VectorSubcoreMesh(core_axis_name="c", subcore_axis_name="s")
_f32 = jnp.float32


def _wid():
    return lax.axis_index("s") * NC + lax.axis_index("c")


def _deg_body(colw, ones, zeros, out, colv, onesv, acc, sem):
    c = lax.axis_index("c")
    s = lax.axis_index("s")
    pltpu.sync_copy(zeros.at[pl.ds(s * ZROWS, ZROWS)], acc.at[pl.ds(s * ZROWS, ZROWS)])
    pltpu.sync_copy(colw.at[_wid()], colv)
    pltpu.sync_copy(ones, onesv)
    plsc.subcore_barrier()
    _scatter_chunks(colv, onesv, acc, sem)
    plsc.subcore_barrier()
    pltpu.sync_copy(acc.at[pl.ds(s * ZROWS, ZROWS)],
                    out.at[pl.ds(c * N_PAD + s * ZROWS, ZROWS)])


def _scatter_chunks(colv, src, acc, sem):

    def chunk(blk, carry):
        j0 = blk * 8
        descs = [pltpu.async_copy(src, acc.at[colv.at[j0 + b]], sem, add=True)
                 for b in range(8)]
        for d in descs:
            d.wait()
        return carry

    lax.fori_loop(0, NCH // 8, chunk, 0)


def _layer_body(roww, colw, ytab, zeros, out, rowv, colv, buf,
                acc, gsem, ssem):
    c = lax.axis_index("c")
    s = lax.axis_index("s")
    pltpu.sync_copy(zeros.at[pl.ds(s * ZROWS, ZROWS)], acc.at[pl.ds(s * ZROWS, ZROWS)])
    pltpu.sync_copy(roww.at[_wid()], rowv)
    pltpu.sync_copy(colw.at[_wid()], colv)
    plsc.subcore_barrier()

    def fire_gather(j, b):
        pltpu.async_copy(ytab.at[rowv.at[j]], buf.at[b], gsem.at[b])

    def wait_gather(j, b):
        pltpu.make_async_copy(ytab.at[rowv.at[j]], buf.at[b], gsem.at[b]).wait()

    def fire_scatter(j, b):
        pltpu.async_copy(buf.at[b], acc.at[colv.at[j]], ssem.at[b], add=True)

    def wait_scatter(j, b):
        pltpu.make_async_copy(buf.at[b], acc.at[colv.at[j]], ssem.at[b]).wait()

    def step(j, jmod):
        b = jmod % K
        wait_gather(j, b)
        fire_scatter(j, b)
        jn, bn = j + L, (jmod + L) % K
        if not isinstance(jn, int) or jn < NCH:
            if not isinstance(jn, int) or jn >= K:
                wait_scatter(jn - K, bn)
            fire_gather(jn, bn)

    for j in range(L):
        fire_gather(j, j % K)

    def outer(blk, carry):
        j0 = blk * K
        for b in range(K):
            step(j0 + b, b)
        return carry

    for b in range(K):
        step(b, b)
    lax.fori_loop(1, NCH // K - 1, outer, 0)
    for b in range(K):
        step(NCH - K + b, b)
    for b in range(K):
        wait_scatter(NCH - K + b, b)
    plsc.subcore_barrier()
    pltpu.sync_copy(acc.at[pl.ds(s * ZROWS, ZROWS)],
                    out.at[pl.ds(c * N_PAD + s * ZROWS, ZROWS)])


_deg_call = pl.kernel(
    _deg_body,
    out_type=jax.ShapeDtypeStruct((NC * N_PAD,), _f32),
    mesh=_mesh,
    compiler_params=pltpu.CompilerParams(use_tc_tiling_on_sc=False),
    scratch_types=[
        pltpu.VMEM((NCH, CH), jnp.int32),
        pltpu.VMEM((CH,), _f32),
        pltpu.VMEM_SHARED((N_PAD,), _f32),
        pltpu.SemaphoreType.DMA,
    ],
)

_layer1d_call = pl.kernel(
    _layer_body,
    out_type=jax.ShapeDtypeStruct((NC * N_PAD,), _f32),
    mesh=_mesh,
    compiler_params=pltpu.CompilerParams(use_tc_tiling_on_sc=False),
    scratch_types=[
        pltpu.VMEM((NCH, CH), jnp.int32),
        pltpu.VMEM((NCH, CH), jnp.int32),
        pltpu.VMEM((K, CH), _f32),
        pltpu.VMEM_SHARED((N_PAD,), _f32),
        pltpu.SemaphoreType.DMA((K,)),
        pltpu.SemaphoreType.DMA((K,)),
    ],
)

_layer_call = pl.kernel(
    _layer_body,
    out_type=jax.ShapeDtypeStruct((NC * N_PAD, D), _f32),
    mesh=_mesh,
    compiler_params=pltpu.CompilerParams(use_tc_tiling_on_sc=False),
    scratch_types=[
        pltpu.VMEM((NCH, CH), jnp.int32),
        pltpu.VMEM((NCH, CH), jnp.int32),
        pltpu.VMEM((K, CH, D), _f32),
        pltpu.VMEM_SHARED((N_PAD, D), _f32),
        pltpu.SemaphoreType.DMA((K,)),
        pltpu.SemaphoreType.DMA((K,)),
    ],
)


def _tc1_body(x_ref, w1_ref, degs_ref, xw_ref, y_ref, dis_ref):
    xw = jnp.dot(x_ref[...], w1_ref[...], preferred_element_type=_f32)
    deg1 = degs_ref[0:N] + degs_ref[N_PAD:N_PAD + N] + 1.0
    dis = jnp.broadcast_to(lax.rsqrt(deg1)[:, None], (N, D))
    xw_ref[...] = xw
    y_ref[...] = dis * xw
    dis_ref[...] = dis


def _tc2_body(acc_ref, xw_ref, dis_ref, w2_ref, b1_ref, y2_ref, hw2_ref):
    a = acc_ref[0:N, :] + acc_ref[N_PAD:N_PAD + N, :]
    dis = dis_ref[...]
    h = dis * a + dis * dis * xw_ref[...] + b1_ref[...]
    h = jnp.maximum(h, 0.0)
    hw2 = jnp.dot(h, w2_ref[...], preferred_element_type=_f32)
    y2_ref[...] = (dis * hw2)[:, 0]
    hw2_ref[...] = jnp.broadcast_to(hw2, (N, D))


def _tc3_body(acc_ref, hw2_ref, dis_ref, b2_ref, out_ref):
    a = acc_ref[0:N] + acc_ref[N_PAD:N_PAD + N]
    dis = dis_ref[...]
    o = dis * a[:, None] + dis * dis * hw2_ref[...] + b2_ref[...]
    out_ref[...] = o[:, 0:1]


_sds = jax.ShapeDtypeStruct

_tc1_call = pl.pallas_call(
    _tc1_body,
    out_shape=(_sds((N, D), _f32), _sds((N, D), _f32), _sds((N, D), _f32)),
)

_tc2_call = pl.pallas_call(
    _tc2_body,
    out_shape=(_sds((N,), _f32), _sds((N, D), _f32)),
)

_tc3_call = pl.pallas_call(
    _tc3_body,
    out_shape=_sds((N, 1), _f32),
)


def kernel(x, edge_index, W1, b1, W2, b2):
    ei = edge_index.astype(jnp.int32)
    pad = jnp.full((2, E_PAD - E), N, dtype=jnp.int32)
    eip = jnp.concatenate([ei, pad], axis=1)
    roww = eip[0].reshape(NW, NCH, CH)
    colw = eip[1].reshape(NW, NCH, CH)

    zeros = jnp.zeros((N_PAD, D), dtype=_f32)
    ones1 = jnp.ones((CH,), dtype=_f32)
    zeros1 = jnp.zeros((N_PAD,), dtype=_f32)

    degs = _deg_call(colw, ones1, zeros1)
    xw1, y1, dis = _tc1_call(x, W1, degs)

    y1p = jnp.concatenate([y1, jnp.zeros((N_PAD - N, D), dtype=_f32)], axis=0)
    acc1 = _layer_call(roww, colw, y1p, zeros)

    y2, hw2 = _tc2_call(acc1, xw1, dis, W2, b1.reshape(1, D))
    y2p = jnp.concatenate([y2, jnp.zeros((N_PAD - N,), dtype=_f32)], axis=0)
    acc2 = _layer1d_call(roww, colw, y2p, zeros1)

    return _tc3_call(acc2, hw2, dis, b2.reshape(1, 1))


# speedup vs baseline: 80.0275x; 1.6494x over previous
import functools

import jax
import jax.numpy as jnp
from jax import lax
from jax.experimental import pallas as pl
from jax.experimental.pallas import tpu as pltpu
from jax.experimental.pallas import tpu_sc as plsc

N = 10000
E = 320000
D = 16

NC = 2
NS = 16
NW = NC * NS
CH = 128
K = 8
L = 4
NCH = 80
EPW = NCH * CH
E_PAD = NW * EPW
N_PAD = 10112
ZROWS = N_PAD // NS

_mesh = plsc.VectorSubcoreMesh(core_axis_name="c", subcore_axis_name="s")
_f32 = jnp.float32


def _wid():
    return lax.axis_index("s") * NC + lax.axis_index("c")


def _deg_body(colw, ones, zeros, out, colv, onesv, acc, sem):
    c = lax.axis_index("c")
    s = lax.axis_index("s")
    pltpu.sync_copy(zeros.at[pl.ds(s * ZROWS, ZROWS)], acc.at[pl.ds(s * ZROWS, ZROWS)])
    pltpu.sync_copy(colw.at[_wid()], colv)
    pltpu.sync_copy(ones, onesv)
    plsc.subcore_barrier()
    _scatter_chunks(colv, onesv, acc, sem)
    plsc.subcore_barrier()
    pltpu.sync_copy(acc.at[pl.ds(s * ZROWS, ZROWS)],
                    out.at[pl.ds(c * N_PAD + s * ZROWS, ZROWS)])


def _scatter_chunks(colv, src, acc, sem):

    def chunk(blk, carry):
        j0 = blk * 8
        descs = [pltpu.async_copy(src, acc.at[colv.at[j0 + b]], sem, add=True)
                 for b in range(8)]
        for d in descs:
            d.wait()
        return carry

    lax.fori_loop(0, NCH // 8, chunk, 0)


def _layer_body(roww, colw, ytab, zeros, out, rowv, colv, buf,
                acc, stab, gsem, ssem):
    c = lax.axis_index("c")
    s = lax.axis_index("s")
    pltpu.sync_copy(zeros.at[pl.ds(s * ZROWS, ZROWS)], acc.at[pl.ds(s * ZROWS, ZROWS)])
    pltpu.sync_copy(ytab.at[pl.ds(s * ZROWS, ZROWS)], stab.at[pl.ds(s * ZROWS, ZROWS)])
    pltpu.sync_copy(roww.at[_wid()], rowv)
    pltpu.sync_copy(colw.at[_wid()], colv)
    plsc.subcore_barrier()
    ytab = stab

    def fire_gather(j, b):
        pltpu.async_copy(ytab.at[rowv.at[j]], buf.at[b], gsem.at[b])

    def wait_gather(j, b):
        pltpu.make_async_copy(ytab.at[rowv.at[j]], buf.at[b], gsem.at[b]).wait()

    def fire_scatter(j, b):
        pltpu.async_copy(buf.at[b], acc.at[colv.at[j]], ssem.at[b], add=True)

    def wait_scatter(j, b):
        pltpu.make_async_copy(buf.at[b], acc.at[colv.at[j]], ssem.at[b]).wait()

    def step(j, jmod):
        b = jmod % K
        wait_gather(j, b)
        fire_scatter(j, b)
        jn, bn = j + L, (jmod + L) % K
        if not isinstance(jn, int) or jn < NCH:
            if not isinstance(jn, int) or jn >= K:
                wait_scatter(jn - K, bn)
            fire_gather(jn, bn)

    for j in range(L):
        fire_gather(j, j % K)

    def outer(blk, carry):
        j0 = blk * K
        for b in range(K):
            step(j0 + b, b)
        return carry

    for b in range(K):
        step(b, b)
    lax.fori_loop(1, NCH // K - 1, outer, 0)
    for b in range(K):
        step(NCH - K + b, b)
    for b in range(K):
        wait_scatter(NCH - K + b, b)
    plsc.subcore_barrier()
    pltpu.sync_copy(acc.at[pl.ds(s * ZROWS, ZROWS)],
                    out.at[pl.ds(c * N_PAD + s * ZROWS, ZROWS)])


_deg_call = pl.kernel(
    _deg_body,
    out_type=jax.ShapeDtypeStruct((NC * N_PAD,), _f32),
    mesh=_mesh,
    compiler_params=pltpu.CompilerParams(use_tc_tiling_on_sc=False),
    scratch_types=[
        pltpu.VMEM((NCH, CH), jnp.int32),
        pltpu.VMEM((CH,), _f32),
        pltpu.VMEM_SHARED((N_PAD,), _f32),
        pltpu.SemaphoreType.DMA,
    ],
)

_layer1d_call = pl.kernel(
    _layer_body,
    out_type=jax.ShapeDtypeStruct((NC * N_PAD,), _f32),
    mesh=_mesh,
    compiler_params=pltpu.CompilerParams(use_tc_tiling_on_sc=False),
    scratch_types=[
        pltpu.VMEM((NCH, CH), jnp.int32),
        pltpu.VMEM((NCH, CH), jnp.int32),
        pltpu.VMEM((K, CH), _f32),
        pltpu.VMEM_SHARED((N_PAD,), _f32),
        pltpu.VMEM_SHARED((N_PAD,), _f32),
        pltpu.SemaphoreType.DMA((K,)),
        pltpu.SemaphoreType.DMA((K,)),
    ],
)

_layer_call = pl.kernel(
    _layer_body,
    out_type=jax.ShapeDtypeStruct((NC * N_PAD, D), _f32),
    mesh=_mesh,
    compiler_params=pltpu.CompilerParams(use_tc_tiling_on_sc=False),
    scratch_types=[
        pltpu.VMEM((NCH, CH), jnp.int32),
        pltpu.VMEM((NCH, CH), jnp.int32),
        pltpu.VMEM((K, CH, D), _f32),
        pltpu.VMEM_SHARED((N_PAD, D), _f32),
        pltpu.VMEM_SHARED((N_PAD, D), _f32),
        pltpu.SemaphoreType.DMA((K,)),
        pltpu.SemaphoreType.DMA((K,)),
    ],
)


def _tc1_body(x_ref, w1_ref, degs_ref, xw_ref, y_ref, dis_ref):
    xw = jnp.dot(x_ref[...], w1_ref[...], preferred_element_type=_f32)
    deg1 = degs_ref[0:N] + degs_ref[N_PAD:N_PAD + N] + 1.0
    dis = jnp.broadcast_to(lax.rsqrt(deg1)[:, None], (N, D))
    xw_ref[...] = xw
    y_ref[...] = dis * xw
    dis_ref[...] = dis


def _tc2_body(acc_ref, xw_ref, dis_ref, w2_ref, b1_ref, y2_ref, hw2_ref):
    a = acc_ref[0:N, :] + acc_ref[N_PAD:N_PAD + N, :]
    dis = dis_ref[...]
    h = dis * a + dis * dis * xw_ref[...] + b1_ref[...]
    h = jnp.maximum(h, 0.0)
    hw2 = jnp.dot(h, w2_ref[...], preferred_element_type=_f32)
    y2_ref[...] = (dis * hw2)[:, 0]
    hw2_ref[...] = jnp.broadcast_to(hw2, (N, D))


def _tc3_body(acc_ref, hw2_ref, dis_ref, b2_ref, out_ref):
    a = acc_ref[0:N] + acc_ref[N_PAD:N_PAD + N]
    dis = dis_ref[...]
    o = dis * a[:, None] + dis * dis * hw2_ref[...] + b2_ref[...]
    out_ref[...] = o[:, 0:1]


_sds = jax.ShapeDtypeStruct

_tc1_call = pl.pallas_call(
    _tc1_body,
    out_shape=(_sds((N, D), _f32), _sds((N, D), _f32), _sds((N, D), _f32)),
)

_tc2_call = pl.pallas_call(
    _tc2_body,
    out_shape=(_sds((N,), _f32), _sds((N, D), _f32)),
)

_tc3_call = pl.pallas_call(
    _tc3_body,
    out_shape=_sds((N, 1), _f32),
)


def kernel(x, edge_index, W1, b1, W2, b2):
    ei = edge_index.astype(jnp.int32)
    pad = jnp.full((2, E_PAD - E), N, dtype=jnp.int32)
    eip = jnp.concatenate([ei, pad], axis=1)
    roww = eip[0].reshape(NW, NCH, CH)
    colw = eip[1].reshape(NW, NCH, CH)

    zeros = jnp.zeros((N_PAD, D), dtype=_f32)
    ones1 = jnp.ones((CH,), dtype=_f32)
    zeros1 = jnp.zeros((N_PAD,), dtype=_f32)

    degs = _deg_call(colw, ones1, zeros1)
    xw1, y1, dis = _tc1_call(x, W1, degs)

    y1p = jnp.concatenate([y1, jnp.zeros((N_PAD - N, D), dtype=_f32)], axis=0)
    acc1 = _layer_call(roww, colw, y1p, zeros)

    y2, hw2 = _tc2_call(acc1, xw1, dis, W2, b1.reshape(1, D))
    y2p = jnp.concatenate([y2, jnp.zeros((N_PAD - N,), dtype=_f32)], axis=0)
    acc2 = _layer1d_call(roww, colw, y2p, zeros1)

    return _tc3_call(acc2, hw2, dis, b2.reshape(1, 1))


# speedup vs baseline: 80.7804x; 1.0094x over previous
import functools

import jax
import jax.numpy as jnp
from jax import lax
from jax.experimental import pallas as pl
from jax.experimental.pallas import tpu as pltpu
from jax.experimental.pallas import tpu_sc as plsc

N = 10000
E = 320000
D = 16

NC = 2
NS = 16
NW = NC * NS
CH = 128
K = 8
L = 4
NCH = 80
EPW = NCH * CH
E_PAD = NW * EPW
N_PAD = 10112
ZROWS = N_PAD // NS

_mesh = plsc.VectorSubcoreMesh(core_axis_name="c", subcore_axis_name="s")
_f32 = jnp.float32


def _wid():
    return lax.axis_index("s") * NC + lax.axis_index("c")


def _deg_body(colw, ones, zeros, out, colv, onesv, acc, sem):
    c = lax.axis_index("c")
    s = lax.axis_index("s")
    pltpu.sync_copy(zeros.at[pl.ds(s * ZROWS, ZROWS)], acc.at[pl.ds(s * ZROWS, ZROWS)])
    pltpu.sync_copy(colw.at[_wid()], colv)
    pltpu.sync_copy(ones, onesv)
    plsc.subcore_barrier()
    _scatter_chunks(colv, onesv, acc, sem)
    plsc.subcore_barrier()
    pltpu.sync_copy(acc.at[pl.ds(s * ZROWS, ZROWS)],
                    out.at[pl.ds(c * N_PAD + s * ZROWS, ZROWS)])


def _scatter_chunks(colv, src, acc, sem):

    def chunk(blk, carry):
        j0 = blk * 8
        descs = [pltpu.async_copy(src, acc.at[colv.at[j0 + b]], sem, add=True)
                 for b in range(8)]
        for d in descs:
            d.wait()
        return carry

    lax.fori_loop(0, NCH // 8, chunk, 0)


def _layer_body(roww, colw, ytab, zeros, out, rowv, colv, buf,
                acc, stab, gsem, ssem):
    c = lax.axis_index("c")
    s = lax.axis_index("s")
    pltpu.sync_copy(zeros.at[pl.ds(s * ZROWS, ZROWS)], acc.at[pl.ds(s * ZROWS, ZROWS)])
    pltpu.sync_copy(ytab.at[pl.ds(s * ZROWS, ZROWS)], stab.at[pl.ds(s * ZROWS, ZROWS)])
    pltpu.sync_copy(roww.at[_wid()], rowv)
    pltpu.sync_copy(colw.at[_wid()], colv)
    plsc.subcore_barrier()
    ytab = stab

    def fire_gather(j, b):
        pltpu.async_copy(ytab.at[rowv.at[j]], buf.at[b], gsem.at[b])

    def wait_gather(j, b):
        pltpu.make_async_copy(ytab.at[rowv.at[j]], buf.at[b], gsem.at[b]).wait()

    def fire_scatter(j, b):
        pltpu.async_copy(buf.at[b], acc.at[colv.at[j]], ssem.at[b], add=True)

    def wait_scatter(j, b):
        pltpu.make_async_copy(buf.at[b], acc.at[colv.at[j]], ssem.at[b]).wait()

    def step(j, jmod):
        b = jmod % K
        wait_gather(j, b)
        fire_scatter(j, b)
        jn, bn = j + L, (jmod + L) % K
        if not isinstance(jn, int) or jn < NCH:
            if not isinstance(jn, int) or jn >= K:
                wait_scatter(jn - K, bn)
            fire_gather(jn, bn)

    for j in range(L):
        fire_gather(j, j % K)

    def outer(blk, carry):
        j0 = blk * K
        for b in range(K):
            step(j0 + b, b)
        return carry

    for b in range(K):
        step(b, b)
    lax.fori_loop(1, NCH // K - 1, outer, 0)
    for b in range(K):
        step(NCH - K + b, b)
    for b in range(K):
        wait_scatter(NCH - K + b, b)
    plsc.subcore_barrier()
    pltpu.sync_copy(acc.at[pl.ds(s * ZROWS, ZROWS)],
                    out.at[pl.ds(c * N_PAD + s * ZROWS, ZROWS)])


_deg_call = pl.kernel(
    _deg_body,
    out_type=jax.ShapeDtypeStruct((NC * N_PAD,), _f32),
    mesh=_mesh,
    compiler_params=pltpu.CompilerParams(use_tc_tiling_on_sc=False),
    scratch_types=[
        pltpu.VMEM((NCH, CH), jnp.int32),
        pltpu.VMEM((CH,), _f32),
        pltpu.VMEM_SHARED((N_PAD,), _f32),
        pltpu.SemaphoreType.DMA,
    ],
)

_layer1d_call = pl.kernel(
    _layer_body,
    out_type=jax.ShapeDtypeStruct((NC * N_PAD,), _f32),
    mesh=_mesh,
    compiler_params=pltpu.CompilerParams(use_tc_tiling_on_sc=False),
    scratch_types=[
        pltpu.VMEM((NCH, CH), jnp.int32),
        pltpu.VMEM((NCH, CH), jnp.int32),
        pltpu.VMEM((K, CH), _f32),
        pltpu.VMEM_SHARED((N_PAD,), _f32),
        pltpu.VMEM_SHARED((N_PAD,), _f32),
        pltpu.SemaphoreType.DMA((K,)),
        pltpu.SemaphoreType.DMA((K,)),
    ],
)

_layer_call = pl.kernel(
    _layer_body,
    out_type=jax.ShapeDtypeStruct((NC * N_PAD, D), _f32),
    mesh=_mesh,
    compiler_params=pltpu.CompilerParams(use_tc_tiling_on_sc=False),
    scratch_types=[
        pltpu.VMEM((NCH, CH), jnp.int32),
        pltpu.VMEM((NCH, CH), jnp.int32),
        pltpu.VMEM((K, CH, D), _f32),
        pltpu.VMEM_SHARED((N_PAD, D), _f32),
        pltpu.VMEM_SHARED((N_PAD, D), _f32),
        pltpu.SemaphoreType.DMA((K,)),
        pltpu.SemaphoreType.DMA((K,)),
    ],
)


def _tc1_body(x_ref, w1_ref, degs_ref, xw_ref, y_ref, dis_ref):
    xw = jnp.dot(x_ref[...], w1_ref[...], preferred_element_type=_f32)
    deg1 = degs_ref[0:N] + degs_ref[N_PAD:N_PAD + N] + 1.0
    dis = jnp.broadcast_to(lax.rsqrt(deg1)[:, None], (N, D))
    xw_ref[...] = xw
    y_ref[0:N, :] = dis * xw
    y_ref[N:N_PAD, :] = jnp.zeros((N_PAD - N, D), _f32)
    dis_ref[...] = dis


def _tc2_body(acc_ref, xw_ref, dis_ref, w2_ref, b1_ref, y2_ref, hw2_ref):
    a = acc_ref[0:N, :] + acc_ref[N_PAD:N_PAD + N, :]
    dis = dis_ref[...]
    h = dis * a + dis * dis * xw_ref[...] + b1_ref[...]
    h = jnp.maximum(h, 0.0)
    hw2 = jnp.dot(h, w2_ref[...], preferred_element_type=_f32)
    y2_ref[0:N] = (dis * hw2)[:, 0]
    y2_ref[N:N_PAD] = jnp.zeros((N_PAD - N,), _f32)
    hw2_ref[...] = jnp.broadcast_to(hw2, (N, D))


def _tc3_body(acc_ref, hw2_ref, dis_ref, b2_ref, out_ref):
    a = acc_ref[0:N] + acc_ref[N_PAD:N_PAD + N]
    dis = dis_ref[...]
    o = dis * a[:, None] + dis * dis * hw2_ref[...] + b2_ref[...]
    out_ref[...] = o[:, 0:1]


_sds = jax.ShapeDtypeStruct

_tc1_call = pl.pallas_call(
    _tc1_body,
    out_shape=(_sds((N, D), _f32), _sds((N_PAD, D), _f32), _sds((N, D), _f32)),
)

_tc2_call = pl.pallas_call(
    _tc2_body,
    out_shape=(_sds((N_PAD,), _f32), _sds((N, D), _f32)),
)

_tc3_call = pl.pallas_call(
    _tc3_body,
    out_shape=_sds((N, 1), _f32),
)


def kernel(x, edge_index, W1, b1, W2, b2):
    ei = edge_index.astype(jnp.int32)
    pad = jnp.full((2, E_PAD - E), N, dtype=jnp.int32)
    eip = jnp.concatenate([ei, pad], axis=1)
    roww = eip[0].reshape(NW, NCH, CH)
    colw = eip[1].reshape(NW, NCH, CH)

    zeros = jnp.zeros((N_PAD, D), dtype=_f32)
    ones1 = jnp.ones((CH,), dtype=_f32)
    zeros1 = jnp.zeros((N_PAD,), dtype=_f32)

    degs = _deg_call(colw, ones1, zeros1)
    xw1, y1p, dis = _tc1_call(x, W1, degs)
    acc1 = _layer_call(roww, colw, y1p, zeros)

    y2p, hw2 = _tc2_call(acc1, xw1, dis, W2, b1.reshape(1, D))
    acc2 = _layer1d_call(roww, colw, y2p, zeros1)

    return _tc3_call(acc2, hw2, dis, b2.reshape(1, 1))


# speedup vs baseline: 86.0871x; 1.0657x over previous
import functools

import jax
import jax.numpy as jnp
from jax import lax
from jax.experimental import pallas as pl
from jax.experimental.pallas import tpu as pltpu
from jax.experimental.pallas import tpu_sc as plsc

N = 10000
E = 320000
D = 16

NC = 2
NS = 16
NW = NC * NS
CH = 128
K = 6
L = 3
NROWS = E // CH
NCH = 78
N_PAD = 10112
ZROWS = N_PAD // NS

_mesh = plsc.VectorSubcoreMesh(core_axis_name="c", subcore_axis_name="s")
_f32 = jnp.float32


def _wid():
    return lax.axis_index("s") * NC + lax.axis_index("c")


def _load_edges(src2d, dstv, w):
    r0 = NCH * w + jnp.minimum(w, 4)
    pltpu.sync_copy(src2d.at[pl.ds(r0, NCH)], dstv.at[pl.ds(0, NCH)])

    @pl.when(w < 4)
    def _():
        pltpu.sync_copy(src2d.at[pl.ds(r0 + NCH, 1)], dstv.at[pl.ds(NCH, 1)])


def _deg_body(colw, ones, zeros, out, colv, onesv, acc, sem):
    c = lax.axis_index("c")
    s = lax.axis_index("s")
    w = _wid()
    pltpu.sync_copy(zeros.at[pl.ds(s * ZROWS, ZROWS)], acc.at[pl.ds(s * ZROWS, ZROWS)])
    _load_edges(colw, colv, w)
    pltpu.sync_copy(ones, onesv)
    plsc.subcore_barrier()
    _scatter_chunks(colv, onesv, acc, sem)

    @pl.when(w < 4)
    def _():
        pltpu.sync_copy(onesv, acc.at[colv.at[NCH]], add=True)

    plsc.subcore_barrier()
    pltpu.sync_copy(acc.at[pl.ds(s * ZROWS, ZROWS)],
                    out.at[pl.ds(c * N_PAD + s * ZROWS, ZROWS)])


def _scatter_chunks(colv, src, acc, sem):

    def chunk(blk, carry):
        j0 = blk * 6
        descs = [pltpu.async_copy(src, acc.at[colv.at[j0 + b]], sem, add=True)
                 for b in range(6)]
        for d in descs:
            d.wait()
        return carry

    lax.fori_loop(0, NCH // 6, chunk, 0)


def _layer_body(roww, colw, ytab, zeros, out, rowv, colv, buf,
                acc, stab, gsem, ssem):
    c = lax.axis_index("c")
    s = lax.axis_index("s")
    pltpu.sync_copy(zeros.at[pl.ds(s * ZROWS, ZROWS)], acc.at[pl.ds(s * ZROWS, ZROWS)])
    pltpu.sync_copy(ytab.at[pl.ds(s * ZROWS, ZROWS)], stab.at[pl.ds(s * ZROWS, ZROWS)])
    w = _wid()
    _load_edges(roww, rowv, w)
    _load_edges(colw, colv, w)
    plsc.subcore_barrier()
    ytab = stab

    def fire_gather(j, b):
        pltpu.async_copy(ytab.at[rowv.at[j]], buf.at[b], gsem.at[b])

    def wait_gather(j, b):
        pltpu.make_async_copy(ytab.at[rowv.at[j]], buf.at[b], gsem.at[b]).wait()

    def fire_scatter(j, b):
        pltpu.async_copy(buf.at[b], acc.at[colv.at[j]], ssem.at[b], add=True)

    def wait_scatter(j, b):
        pltpu.make_async_copy(buf.at[b], acc.at[colv.at[j]], ssem.at[b]).wait()

    def step(j, jmod):
        b = jmod % K
        wait_gather(j, b)
        fire_scatter(j, b)
        jn, bn = j + L, (jmod + L) % K
        if not isinstance(jn, int) or jn < NCH:
            if not isinstance(jn, int) or jn >= K:
                wait_scatter(jn - K, bn)
            fire_gather(jn, bn)

    for j in range(L):
        fire_gather(j, j % K)

    def outer(blk, carry):
        j0 = blk * K
        for b in range(K):
            step(j0 + b, b)
        return carry

    for b in range(K):
        step(b, b)
    lax.fori_loop(1, NCH // K - 1, outer, 0)
    for b in range(K):
        step(NCH - K + b, b)
    for b in range(K):
        wait_scatter(NCH - K + b, b)

    @pl.when(w < 4)
    def _():
        pltpu.sync_copy(ytab.at[rowv.at[NCH]], buf.at[0])
        pltpu.sync_copy(buf.at[0], acc.at[colv.at[NCH]], add=True)

    plsc.subcore_barrier()
    pltpu.sync_copy(acc.at[pl.ds(s * ZROWS, ZROWS)],
                    out.at[pl.ds(c * N_PAD + s * ZROWS, ZROWS)])


_deg_call = pl.kernel(
    _deg_body,
    out_type=jax.ShapeDtypeStruct((NC * N_PAD,), _f32),
    mesh=_mesh,
    compiler_params=pltpu.CompilerParams(use_tc_tiling_on_sc=False),
    scratch_types=[
        pltpu.VMEM((NCH + 1, CH), jnp.int32),
        pltpu.VMEM((CH,), _f32),
        pltpu.VMEM_SHARED((N_PAD,), _f32),
        pltpu.SemaphoreType.DMA,
    ],
)

_layer1d_call = pl.kernel(
    _layer_body,
    out_type=jax.ShapeDtypeStruct((NC * N_PAD,), _f32),
    mesh=_mesh,
    compiler_params=pltpu.CompilerParams(use_tc_tiling_on_sc=False),
    scratch_types=[
        pltpu.VMEM((NCH + 1, CH), jnp.int32),
        pltpu.VMEM((NCH + 1, CH), jnp.int32),
        pltpu.VMEM((K, CH), _f32),
        pltpu.VMEM_SHARED((N_PAD,), _f32),
        pltpu.VMEM_SHARED((N_PAD,), _f32),
        pltpu.SemaphoreType.DMA((K,)),
        pltpu.SemaphoreType.DMA((K,)),
    ],
)

_layer_call = pl.kernel(
    _layer_body,
    out_type=jax.ShapeDtypeStruct((NC * N_PAD, D), _f32),
    mesh=_mesh,
    compiler_params=pltpu.CompilerParams(use_tc_tiling_on_sc=False),
    scratch_types=[
        pltpu.VMEM((NCH + 1, CH), jnp.int32),
        pltpu.VMEM((NCH + 1, CH), jnp.int32),
        pltpu.VMEM((K, CH, D), _f32),
        pltpu.VMEM_SHARED((N_PAD, D), _f32),
        pltpu.VMEM_SHARED((N_PAD, D), _f32),
        pltpu.SemaphoreType.DMA((K,)),
        pltpu.SemaphoreType.DMA((K,)),
    ],
)


def _tca_body(x_ref, w1_ref, xw_ref):
    xw_ref[...] = jnp.dot(x_ref[...], w1_ref[...], preferred_element_type=_f32)


def _tcb_body(degs_ref, xw_ref, y_ref, dis_ref):
    deg1 = degs_ref[0:N] + degs_ref[N_PAD:N_PAD + N] + 1.0
    dis = jnp.broadcast_to(lax.rsqrt(deg1)[:, None], (N, D))
    y_ref[0:N, :] = dis * xw_ref[...]
    y_ref[N:N_PAD, :] = jnp.zeros((N_PAD - N, D), _f32)
    dis_ref[...] = dis


def _tc2_body(acc_ref, xw_ref, dis_ref, w2_ref, b1_ref, y2_ref, hw2_ref):
    a = acc_ref[0:N, :] + acc_ref[N_PAD:N_PAD + N, :]
    dis = dis_ref[...]
    h = dis * a + dis * dis * xw_ref[...] + b1_ref[...]
    h = jnp.maximum(h, 0.0)
    hw2 = jnp.dot(h, w2_ref[...], preferred_element_type=_f32)
    y2_ref[0:N] = (dis * hw2)[:, 0]
    y2_ref[N:N_PAD] = jnp.zeros((N_PAD - N,), _f32)
    hw2_ref[...] = jnp.broadcast_to(hw2, (N, D))


def _tc3_body(acc_ref, hw2_ref, dis_ref, b2_ref, out_ref):
    a = acc_ref[0:N] + acc_ref[N_PAD:N_PAD + N]
    dis = dis_ref[...]
    o = dis * a[:, None] + dis * dis * hw2_ref[...] + b2_ref[...]
    out_ref[...] = o[:, 0:1]


_sds = jax.ShapeDtypeStruct

_tca_call = pl.pallas_call(
    _tca_body,
    out_shape=_sds((N, D), _f32),
)

_tcb_call = pl.pallas_call(
    _tcb_body,
    out_shape=(_sds((N_PAD, D), _f32), _sds((N, D), _f32)),
)

_tc2_call = pl.pallas_call(
    _tc2_body,
    out_shape=(_sds((N_PAD,), _f32), _sds((N, D), _f32)),
)

_tc3_call = pl.pallas_call(
    _tc3_body,
    out_shape=_sds((N, 1), _f32),
)


def kernel(x, edge_index, W1, b1, W2, b2):
    ei = edge_index.astype(jnp.int32)
    roww = ei[0].reshape(NROWS, CH)
    colw = ei[1].reshape(NROWS, CH)

    zeros = jnp.zeros((N_PAD, D), dtype=_f32)
    ones1 = jnp.ones((CH,), dtype=_f32)
    zeros1 = jnp.zeros((N_PAD,), dtype=_f32)

    degs = _deg_call(colw, ones1, zeros1)
    xw1 = _tca_call(x, W1)
    y1p, dis = _tcb_call(degs, xw1)
    acc1 = _layer_call(roww, colw, y1p, zeros)

    y2p, hw2 = _tc2_call(acc1, xw1, dis, W2, b1.reshape(1, D))
    acc2 = _layer1d_call(roww, colw, y2p, zeros1)

    return _tc3_call(acc2, hw2, dis, b2.reshape(1, 1))


# speedup vs baseline: 93.5671x; 1.0869x over previous
import functools

import jax
import jax.numpy as jnp
from jax import lax
from jax.experimental import pallas as pl
from jax.experimental.pallas import tpu as pltpu
from jax.experimental.pallas import tpu_sc as plsc

N = 10000
E = 320000
D = 16

NC = 2
NS = 16
NW = NC * NS
CH = 128
K = 6
L = 3
NROWS = E // CH
NCH = 78
N_PAD = 10112
ZROWS = N_PAD // NS

_mesh = plsc.VectorSubcoreMesh(core_axis_name="c", subcore_axis_name="s")
_f32 = jnp.float32


def _wid():
    return lax.axis_index("s") * NC + lax.axis_index("c")


def _load_edges(ei3, d, dstv, w):
    r0 = NCH * w + jnp.minimum(w, 4)
    pltpu.sync_copy(ei3.at[d, pl.ds(r0, NCH)], dstv.at[pl.ds(0, NCH)])

    @pl.when(w < 4)
    def _():
        pltpu.sync_copy(ei3.at[d, pl.ds(r0 + NCH, 1)], dstv.at[pl.ds(NCH, 1)])


def _deg_body(ei3, ones, zeros, out, colv, onesv, acc, sem):
    c = lax.axis_index("c")
    s = lax.axis_index("s")
    w = _wid()
    pltpu.sync_copy(zeros.at[pl.ds(s * ZROWS, ZROWS)], acc.at[pl.ds(s * ZROWS, ZROWS)])
    _load_edges(ei3, 1, colv, w)
    pltpu.sync_copy(ones, onesv)
    plsc.subcore_barrier()
    _scatter_chunks(colv, onesv, acc, sem)

    @pl.when(w < 4)
    def _():
        pltpu.sync_copy(onesv, acc.at[colv.at[NCH]], add=True)

    plsc.subcore_barrier()
    pltpu.sync_copy(acc.at[pl.ds(s * ZROWS, ZROWS)],
                    out.at[pl.ds(c * N_PAD + s * ZROWS, ZROWS)])


def _scatter_chunks(colv, src, acc, sem):

    def chunk(blk, carry):
        j0 = blk * 6
        descs = [pltpu.async_copy(src, acc.at[colv.at[j0 + b]], sem, add=True)
                 for b in range(6)]
        for d in descs:
            d.wait()
        return carry

    lax.fori_loop(0, NCH // 6, chunk, 0)


def _layer_body(ei3, ytab, zeros, out, rowv, colv, buf,
                acc, stab, gsem, ssem):
    c = lax.axis_index("c")
    s = lax.axis_index("s")
    pltpu.sync_copy(zeros.at[pl.ds(s * ZROWS, ZROWS)], acc.at[pl.ds(s * ZROWS, ZROWS)])
    pltpu.sync_copy(ytab.at[pl.ds(s * ZROWS, ZROWS)], stab.at[pl.ds(s * ZROWS, ZROWS)])
    w = _wid()
    _load_edges(ei3, 0, rowv, w)
    _load_edges(ei3, 1, colv, w)
    plsc.subcore_barrier()
    ytab = stab

    def fire_gather(j, b):
        pltpu.async_copy(ytab.at[rowv.at[j]], buf.at[b], gsem.at[b])

    def wait_gather(j, b):
        pltpu.make_async_copy(ytab.at[rowv.at[j]], buf.at[b], gsem.at[b]).wait()

    def fire_scatter(j, b):
        pltpu.async_copy(buf.at[b], acc.at[colv.at[j]], ssem.at[b], add=True)

    def wait_scatter(j, b):
        pltpu.make_async_copy(buf.at[b], acc.at[colv.at[j]], ssem.at[b]).wait()

    def step(j, jmod):
        b = jmod % K
        wait_gather(j, b)
        fire_scatter(j, b)
        jn, bn = j + L, (jmod + L) % K
        if not isinstance(jn, int) or jn < NCH:
            if not isinstance(jn, int) or jn >= K:
                wait_scatter(jn - K, bn)
            fire_gather(jn, bn)

    for j in range(L):
        fire_gather(j, j % K)

    def outer(blk, carry):
        j0 = blk * K
        for b in range(K):
            step(j0 + b, b)
        return carry

    for b in range(K):
        step(b, b)
    lax.fori_loop(1, NCH // K - 1, outer, 0)
    for b in range(K):
        step(NCH - K + b, b)
    for b in range(K):
        wait_scatter(NCH - K + b, b)

    @pl.when(w < 4)
    def _():
        pltpu.sync_copy(ytab.at[rowv.at[NCH]], buf.at[0])
        pltpu.sync_copy(buf.at[0], acc.at[colv.at[NCH]], add=True)

    plsc.subcore_barrier()
    pltpu.sync_copy(acc.at[pl.ds(s * ZROWS, ZROWS)],
                    out.at[pl.ds(c * N_PAD + s * ZROWS, ZROWS)])


_deg_call = pl.kernel(
    _deg_body,
    out_type=jax.ShapeDtypeStruct((NC * N_PAD,), _f32),
    mesh=_mesh,
    compiler_params=pltpu.CompilerParams(use_tc_tiling_on_sc=False),
    scratch_types=[
        pltpu.VMEM((NCH + 1, CH), jnp.int32),
        pltpu.VMEM((CH,), _f32),
        pltpu.VMEM_SHARED((N_PAD,), _f32),
        pltpu.SemaphoreType.DMA,
    ],
)

_layer1d_call = pl.kernel(
    _layer_body,
    out_type=jax.ShapeDtypeStruct((NC * N_PAD,), _f32),
    mesh=_mesh,
    compiler_params=pltpu.CompilerParams(use_tc_tiling_on_sc=False),
    scratch_types=[
        pltpu.VMEM((NCH + 1, CH), jnp.int32),
        pltpu.VMEM((NCH + 1, CH), jnp.int32),
        pltpu.VMEM((K, CH), _f32),
        pltpu.VMEM_SHARED((N_PAD,), _f32),
        pltpu.VMEM_SHARED((N_PAD,), _f32),
        pltpu.SemaphoreType.DMA((K,)),
        pltpu.SemaphoreType.DMA((K,)),
    ],
)

_layer_call = pl.kernel(
    _layer_body,
    out_type=jax.ShapeDtypeStruct((NC * N_PAD, D), _f32),
    mesh=_mesh,
    compiler_params=pltpu.CompilerParams(use_tc_tiling_on_sc=False),
    scratch_types=[
        pltpu.VMEM((NCH + 1, CH), jnp.int32),
        pltpu.VMEM((NCH + 1, CH), jnp.int32),
        pltpu.VMEM((K, CH, D), _f32),
        pltpu.VMEM_SHARED((N_PAD, D), _f32),
        pltpu.VMEM_SHARED((N_PAD, D), _f32),
        pltpu.SemaphoreType.DMA((K,)),
        pltpu.SemaphoreType.DMA((K,)),
    ],
)


def _tca_body(x_ref, w1_ref, xw_ref):
    xw_ref[...] = jnp.dot(x_ref[...], w1_ref[...], preferred_element_type=_f32)


def _tcb_body(degs_ref, xw_ref, y_ref, dis_ref):
    deg1 = degs_ref[0:N] + degs_ref[N_PAD:N_PAD + N] + 1.0
    dis = jnp.broadcast_to(lax.rsqrt(deg1)[:, None], (N, D))
    y_ref[0:N, :] = dis * xw_ref[...]
    y_ref[N:N_PAD, :] = jnp.zeros((N_PAD - N, D), _f32)
    dis_ref[...] = dis


def _tc2_body(acc_ref, xw_ref, dis_ref, w2_ref, b1_ref, y2_ref, hw2_ref):
    a = acc_ref[0:N, :] + acc_ref[N_PAD:N_PAD + N, :]
    dis = dis_ref[...]
    h = dis * a + dis * dis * xw_ref[...] + b1_ref[...]
    h = jnp.maximum(h, 0.0)
    hw2 = jnp.dot(h, w2_ref[...], preferred_element_type=_f32)
    y2_ref[0:N] = (dis * hw2)[:, 0]
    y2_ref[N:N_PAD] = jnp.zeros((N_PAD - N,), _f32)
    hw2_ref[...] = jnp.broadcast_to(hw2, (N, D))


def _tc3_body(acc_ref, hw2_ref, dis_ref, b2_ref, out_ref):
    a = acc_ref[0:N] + acc_ref[N_PAD:N_PAD + N]
    dis = dis_ref[...]
    o = dis * a[:, None] + dis * dis * hw2_ref[...] + b2_ref[...]
    out_ref[...] = o[:, 0:1]


_sds = jax.ShapeDtypeStruct

_tca_call = pl.pallas_call(
    _tca_body,
    out_shape=_sds((N, D), _f32),
)

_tcb_call = pl.pallas_call(
    _tcb_body,
    out_shape=(_sds((N_PAD, D), _f32), _sds((N, D), _f32)),
)

_tc2_call = pl.pallas_call(
    _tc2_body,
    out_shape=(_sds((N_PAD,), _f32), _sds((N, D), _f32)),
)

_tc3_call = pl.pallas_call(
    _tc3_body,
    out_shape=_sds((N, 1), _f32),
)


def kernel(x, edge_index, W1, b1, W2, b2):
    ei3 = edge_index.astype(jnp.int32).reshape(2, NROWS, CH)

    zeros = jnp.zeros((N_PAD, D), dtype=_f32)
    ones1 = jnp.ones((CH,), dtype=_f32)
    zeros1 = jnp.zeros((N_PAD,), dtype=_f32)

    degs = _deg_call(ei3, ones1, zeros1)
    xw1 = _tca_call(x, W1)
    y1p, dis = _tcb_call(degs, xw1)
    acc1 = _layer_call(ei3, y1p, zeros)

    y2p, hw2 = _tc2_call(acc1, xw1, dis, W2, b1.reshape(1, D))
    acc2 = _layer1d_call(ei3, y2p, zeros1)

    return _tc3_call(acc2, hw2, dis, b2.reshape(1, 1))


# speedup vs baseline: 98.0357x; 1.0478x over previous
import functools

import jax
import jax.numpy as jnp
from jax import lax
from jax.experimental import pallas as pl
from jax.experimental.pallas import tpu as pltpu
from jax.experimental.pallas import tpu_sc as plsc

N = 10000
E = 320000
D = 16

NC = 2
NS = 16
NW = NC * NS
CH = 128
K = 6
L = 3
NROWS = E // CH
NCH = 78
N_PAD = 10112
ZROWS = N_PAD // NS

_mesh = plsc.VectorSubcoreMesh(core_axis_name="c", subcore_axis_name="s")
_f32 = jnp.float32


def _wid():
    return lax.axis_index("s") * NC + lax.axis_index("c")


def _load_edges(ei3, d, dstv, w):
    r0 = NCH * w + jnp.minimum(w, 4)
    pltpu.sync_copy(ei3.at[d, pl.ds(r0, NCH)], dstv.at[pl.ds(0, NCH)])

    @pl.when(w < 4)
    def _():
        pltpu.sync_copy(ei3.at[d, pl.ds(r0 + NCH, 1)], dstv.at[pl.ds(NCH, 1)])


def _deg_body(ei3, ones, zeros, out, colv, onesv, acc, sem):
    c = lax.axis_index("c")
    s = lax.axis_index("s")
    w = _wid()
    pltpu.sync_copy(zeros.at[pl.ds(s * ZROWS, ZROWS)], acc.at[pl.ds(s * ZROWS, ZROWS)])
    _load_edges(ei3, 1, colv, w)
    pltpu.sync_copy(ones, onesv)
    plsc.subcore_barrier()
    _scatter_chunks(colv, onesv, acc, sem)

    @pl.when(w < 4)
    def _():
        pltpu.sync_copy(onesv, acc.at[colv.at[NCH]], add=True)

    plsc.subcore_barrier()
    pltpu.sync_copy(acc.at[pl.ds(s * ZROWS, ZROWS)],
                    out.at[pl.ds(c * N_PAD + s * ZROWS, ZROWS)])


def _scatter_chunks(colv, src, acc, sem):

    def chunk(blk, carry):
        j0 = blk * 6
        descs = [pltpu.async_copy(src, acc.at[colv.at[j0 + b]], sem, add=True)
                 for b in range(6)]
        for d in descs:
            d.wait()
        return carry

    lax.fori_loop(0, NCH // 6, chunk, 0)


def _layer_body(ei3, ytab, zeros, out, rowv, colv, buf,
                acc, stab, gsem, ssem):
    c = lax.axis_index("c")
    s = lax.axis_index("s")
    pltpu.sync_copy(zeros.at[pl.ds(s * ZROWS, ZROWS)], acc.at[pl.ds(s * ZROWS, ZROWS)])
    pltpu.sync_copy(ytab.at[pl.ds(s * ZROWS, ZROWS)], stab.at[pl.ds(s * ZROWS, ZROWS)])
    w = _wid()
    _load_edges(ei3, 0, rowv, w)
    _load_edges(ei3, 1, colv, w)
    plsc.subcore_barrier()
    ytab = stab

    def fire_gather(j, b):
        pltpu.async_copy(ytab.at[rowv.at[j]], buf.at[b], gsem.at[b])

    def wait_gather(j, b):
        pltpu.make_async_copy(ytab.at[rowv.at[j]], buf.at[b], gsem.at[b]).wait()

    def fire_scatter(j, b):
        pltpu.async_copy(buf.at[b], acc.at[colv.at[j]], ssem.at[b], add=True)

    def wait_scatter(j, b):
        pltpu.make_async_copy(buf.at[b], acc.at[colv.at[j]], ssem.at[b]).wait()

    def step(j, jmod):
        b = jmod % K
        wait_gather(j, b)
        fire_scatter(j, b)
        jn, bn = j + L, (jmod + L) % K
        if not isinstance(jn, int) or jn < NCH:
            if not isinstance(jn, int) or jn >= K:
                wait_scatter(jn - K, bn)
            fire_gather(jn, bn)

    for j in range(L):
        fire_gather(j, j % K)

    def outer(blk, carry):
        j0 = blk * K
        for b in range(K):
            step(j0 + b, b)
        return carry

    for b in range(K):
        step(b, b)
    lax.fori_loop(1, NCH // K - 1, outer, 0)
    for b in range(K):
        step(NCH - K + b, b)
    for b in range(K):
        wait_scatter(NCH - K + b, b)

    @pl.when(w < 4)
    def _():
        pltpu.sync_copy(ytab.at[rowv.at[NCH]], buf.at[0])
        pltpu.sync_copy(buf.at[0], acc.at[colv.at[NCH]], add=True)

    plsc.subcore_barrier()
    pltpu.sync_copy(acc.at[pl.ds(s * ZROWS, ZROWS)],
                    out.at[pl.ds(c * N_PAD + s * ZROWS, ZROWS)])


_deg_call = pl.kernel(
    _deg_body,
    out_type=jax.ShapeDtypeStruct((NC * N_PAD,), _f32),
    mesh=_mesh,
    compiler_params=pltpu.CompilerParams(use_tc_tiling_on_sc=False),
    scratch_types=[
        pltpu.VMEM((NCH + 1, CH), jnp.int32),
        pltpu.VMEM((CH,), _f32),
        pltpu.VMEM_SHARED((N_PAD,), _f32),
        pltpu.SemaphoreType.DMA,
    ],
)

_layer1d_call = pl.kernel(
    _layer_body,
    out_type=jax.ShapeDtypeStruct((NC * N_PAD,), _f32),
    mesh=_mesh,
    compiler_params=pltpu.CompilerParams(use_tc_tiling_on_sc=False),
    scratch_types=[
        pltpu.VMEM((NCH + 1, CH), jnp.int32),
        pltpu.VMEM((NCH + 1, CH), jnp.int32),
        pltpu.VMEM((K, CH), _f32),
        pltpu.VMEM_SHARED((N_PAD,), _f32),
        pltpu.VMEM_SHARED((N_PAD,), _f32),
        pltpu.SemaphoreType.DMA((K,)),
        pltpu.SemaphoreType.DMA((K,)),
    ],
)

_layer_call = pl.kernel(
    _layer_body,
    out_type=jax.ShapeDtypeStruct((NC * N_PAD, D), _f32),
    mesh=_mesh,
    compiler_params=pltpu.CompilerParams(use_tc_tiling_on_sc=False),
    scratch_types=[
        pltpu.VMEM((NCH + 1, CH), jnp.int32),
        pltpu.VMEM((NCH + 1, CH), jnp.int32),
        pltpu.VMEM((K, CH, D), _f32),
        pltpu.VMEM_SHARED((N_PAD, D), _f32),
        pltpu.VMEM_SHARED((N_PAD, D), _f32),
        pltpu.SemaphoreType.DMA((K,)),
        pltpu.SemaphoreType.DMA((K,)),
    ],
)


def _tca_body(x_ref, w1_ref, xw_ref):
    xw_ref[...] = jnp.dot(x_ref[...], w1_ref[...], preferred_element_type=_f32)


def _tcb_body(degs_ref, xw_ref, y_ref, dis_ref, dis1_ref):
    deg1 = degs_ref[0:N] + degs_ref[N_PAD:N_PAD + N] + 1.0
    dis1 = lax.rsqrt(deg1)
    dis = jnp.broadcast_to(dis1[:, None], (N, D))
    y_ref[0:N, :] = dis * xw_ref[...]
    y_ref[N:N_PAD, :] = jnp.zeros((N_PAD - N, D), _f32)
    dis_ref[...] = dis
    dis1_ref[...] = dis1


def _tc2_body(acc_ref, xw_ref, dis_ref, w2_ref, b1_ref, y2_ref, hw2_ref):
    a = acc_ref[0:N, :] + acc_ref[N_PAD:N_PAD + N, :]
    dis = dis_ref[...]
    h = dis * (a + dis * xw_ref[...]) + b1_ref[...]
    h = jnp.maximum(h, 0.0)
    hw2 = jnp.sum(h * w2_ref[...], axis=1)
    dis1 = dis[:, 0]
    y2_ref[0:N] = dis1 * hw2
    y2_ref[N:N_PAD] = jnp.zeros((N_PAD - N,), _f32)
    hw2_ref[...] = hw2


def _tc3_body(acc_ref, hw2_ref, dis1_ref, b2_ref, out_ref):
    a = acc_ref[0:N] + acc_ref[N_PAD:N_PAD + N]
    dis1 = dis1_ref[...]
    out_ref[...] = dis1 * a + dis1 * dis1 * hw2_ref[...] + b2_ref[...]


_sds = jax.ShapeDtypeStruct

_tca_call = pl.pallas_call(
    _tca_body,
    out_shape=_sds((N, D), _f32),
)

_tcb_call = pl.pallas_call(
    _tcb_body,
    out_shape=(_sds((N_PAD, D), _f32), _sds((N, D), _f32), _sds((N,), _f32)),
)

_tc2_call = pl.pallas_call(
    _tc2_body,
    out_shape=(_sds((N_PAD,), _f32), _sds((N,), _f32)),
)

_tc3_call = pl.pallas_call(
    _tc3_body,
    out_shape=_sds((N,), _f32),
)


def kernel(x, edge_index, W1, b1, W2, b2):
    ei3 = edge_index.astype(jnp.int32).reshape(2, NROWS, CH)

    zeros = jnp.zeros((N_PAD, D), dtype=_f32)
    ones1 = jnp.ones((CH,), dtype=_f32)
    zeros1 = jnp.zeros((N_PAD,), dtype=_f32)

    degs = _deg_call(ei3, ones1, zeros1)
    xw1 = _tca_call(x, W1)
    y1p, dis, dis1 = _tcb_call(degs, xw1)
    acc1 = _layer_call(ei3, y1p, zeros)

    y2p, hw2 = _tc2_call(acc1, xw1, dis, W2.reshape(1, D), b1.reshape(1, D))
    acc2 = _layer1d_call(ei3, y2p, zeros1)

    out1 = _tc3_call(acc2, hw2, dis1, b2)
    return out1.reshape(N, 1)
